# baseline (speedup 1.0000x reference)
import functools

import jax
import jax.numpy as jnp
from jax import lax
from jax.experimental import pallas as pl
from jax.experimental.pallas import tpu as pltpu
from jax.experimental.pallas import tpu_sc as plsc

B = 16384
F = 26
NC = 2
NS = 16
NW = NC * NS
RPW = B // NW
IDXW = RPW * F
LANES = 16

_mesh = plsc.VectorSubcoreMesh(core_axis_name="c", subcore_axis_name="s")


@functools.partial(
    pl.kernel,
    mesh=_mesh,
    compiler_params=pltpu.CompilerParams(needs_layout_passes=False),
    out_type=jax.ShapeDtypeStruct((B,), jnp.float32),
    scratch_types=[
        pltpu.VMEM((IDXW,), jnp.int32),
        pltpu.VMEM((IDXW,), jnp.float32),
        pltpu.VMEM((RPW,), jnp.float32),
        pltpu.SemaphoreType.DMA,
    ],
)
def _sc_gather_sum(w_hbm, w2_hbm, idx_hbm, out_hbm, idx_v, vals_v, acc_v, sem):
    wid = lax.axis_index("s") * NC + lax.axis_index("c")
    base = wid * IDXW
    pltpu.sync_copy(idx_hbm.at[pl.ds(base, IDXW)], idx_v)
    pltpu.async_copy(w_hbm.at[idx_v], vals_v, sem).wait()
    for c in range(RPW // LANES):
        col = c * LANES
        a = vals_v[pl.ds(col, LANES)]
        for j in range(1, F):
            a = a + vals_v[pl.ds(j * RPW + col, LANES)]
        acc_v[pl.ds(col, LANES)] = a
    pltpu.sync_copy(acc_v, out_hbm.at[pl.ds(wid * RPW, RPW)])


def _tc_body(b_ref, xw_ref, y_ref, yprob_ref, loss_ref):
    logits = xw_ref[...] + b_ref[0, 0]
    yprob_ref[...] = 1.0 / (1.0 + jnp.exp(-logits))
    ce = (
        jnp.maximum(logits, 0.0)
        - logits * y_ref[...]
        + jnp.log(1.0 + jnp.exp(-jnp.abs(logits)))
    )
    loss_ref[0, 0] = jnp.sum(ce) * (1.0 / B)


_tc_tail = pl.pallas_call(
    _tc_body,
    out_shape=(
        jax.ShapeDtypeStruct((B // 128, 128), jnp.float32),
        jax.ShapeDtypeStruct((1, 1), jnp.float32),
    ),
    in_specs=[
        pl.BlockSpec(memory_space=pltpu.SMEM),
        pl.BlockSpec(),
        pl.BlockSpec(),
    ],
    out_specs=(
        pl.BlockSpec(),
        pl.BlockSpec(memory_space=pltpu.SMEM),
    ),
)


def kernel(indices, values, y, w, b):
    idx_perm = indices.reshape(NW, RPW, F).transpose(0, 2, 1).reshape(-1)
    xw = _sc_gather_sum(w.T.reshape(-1), w.reshape(1000, 1000), idx_perm)
    yprob, loss = _tc_tail(
        b.reshape(1, 1), xw.reshape(B // 128, 128), y.reshape(B // 128, 128)
    )
    return yprob.reshape(-1), loss[0, 0]


# speedup vs baseline: 1.0372x; 1.0372x over previous
import functools

import jax
import jax.numpy as jnp
from jax import lax
from jax.experimental import pallas as pl
from jax.experimental.pallas import tpu as pltpu
from jax.experimental.pallas import tpu_sc as plsc

B = 16384
F = 26
NC = 2
NS = 16
NW = NC * NS
RPW = B // NW
IDXW = RPW * F
LANES = 16

_mesh = plsc.VectorSubcoreMesh(core_axis_name="c", subcore_axis_name="s")

_DROWS = 40


@functools.partial(
    pl.kernel,
    mesh=_mesh,
    compiler_params=pltpu.CompilerParams(needs_layout_passes=False),
    out_type=jax.ShapeDtypeStruct((1000 * 1000,), jnp.float32),
    scratch_types=[
        pltpu.VMEM((_DROWS, 1000), jnp.float32),
        pltpu.SemaphoreType.DMA,
        pltpu.SemaphoreType.DMA,
    ],
)
def _sc_detile(w2_hbm, flat_hbm, buf_v, sem_in, sem_out):
    wid = lax.axis_index("s") * NC + lax.axis_index("c")

    @pl.when(wid < 1000 // _DROWS)
    def _():
        base_row = wid * _DROWS
        pltpu.async_copy(w2_hbm.at[pl.ds(base_row, _DROWS), :], buf_v, sem_in).wait()
        descs = [
            pltpu.async_copy(
                buf_v.at[j],
                flat_hbm.at[pl.ds((base_row + j) * 1000, 1000)],
                sem_out,
            )
            for j in range(_DROWS)
        ]
        for d in descs:
            d.wait()


@functools.partial(
    pl.kernel,
    mesh=_mesh,
    compiler_params=pltpu.CompilerParams(needs_layout_passes=False),
    out_type=jax.ShapeDtypeStruct((B,), jnp.float32),
    scratch_types=[
        pltpu.VMEM((IDXW,), jnp.int32),
        pltpu.VMEM((IDXW,), jnp.float32),
        pltpu.VMEM((RPW,), jnp.float32),
        pltpu.SemaphoreType.DMA,
    ],
)
def _sc_gather_sum(w_hbm, idx_hbm, out_hbm, idx_v, vals_v, acc_v, sem):
    wid = lax.axis_index("s") * NC + lax.axis_index("c")
    base = wid * IDXW
    pltpu.sync_copy(idx_hbm.at[pl.ds(base, IDXW)], idx_v)
    pltpu.async_copy(w_hbm.at[idx_v], vals_v, sem).wait()
    for c in range(RPW // LANES):
        col = c * LANES
        a = vals_v[pl.ds(col, LANES)]
        for j in range(1, F):
            a = a + vals_v[pl.ds(j * RPW + col, LANES)]
        acc_v[pl.ds(col, LANES)] = a
    pltpu.sync_copy(acc_v, out_hbm.at[pl.ds(wid * RPW, RPW)])


def _tc_body(b_ref, xw_ref, y_ref, yprob_ref, loss_ref):
    logits = xw_ref[...] + b_ref[0, 0]
    yprob_ref[...] = 1.0 / (1.0 + jnp.exp(-logits))
    ce = (
        jnp.maximum(logits, 0.0)
        - logits * y_ref[...]
        + jnp.log(1.0 + jnp.exp(-jnp.abs(logits)))
    )
    loss_ref[0, 0] = jnp.sum(ce) * (1.0 / B)


_tc_tail = pl.pallas_call(
    _tc_body,
    out_shape=(
        jax.ShapeDtypeStruct((B // 128, 128), jnp.float32),
        jax.ShapeDtypeStruct((1, 1), jnp.float32),
    ),
    in_specs=[
        pl.BlockSpec(memory_space=pltpu.SMEM),
        pl.BlockSpec(),
        pl.BlockSpec(),
    ],
    out_specs=(
        pl.BlockSpec(),
        pl.BlockSpec(memory_space=pltpu.SMEM),
    ),
)


def kernel(indices, values, y, w, b):
    idx_perm = indices.reshape(NW, RPW, F).transpose(0, 2, 1).reshape(-1)
    w128 = jax.lax.optimization_barrier(jnp.pad(w, ((0, 64), (0, 0))).reshape(7813, 128))
    xw = _sc_gather_sum(w128.reshape(-1), idx_perm)
    yprob, loss = _tc_tail(
        b.reshape(1, 1), xw.reshape(B // 128, 128), y.reshape(B // 128, 128)
    )
    return yprob.reshape(-1), loss[0, 0]


# speedup vs baseline: 1.4591x; 1.4069x over previous
import functools

import jax
import jax.numpy as jnp
from jax import lax
from jax.experimental import pallas as pl
from jax.experimental.pallas import tpu as pltpu
from jax.experimental.pallas import tpu_sc as plsc

B = 16384
F = 26
NC = 2
NS = 16
NW = NC * NS
RPW = B // NW
IDXW = RPW * F
LANES = 16

_mesh = plsc.VectorSubcoreMesh(core_axis_name="c", subcore_axis_name="s")

@functools.partial(
    pl.kernel,
    mesh=_mesh,
    compiler_params=pltpu.CompilerParams(needs_layout_passes=False),
    out_type=jax.ShapeDtypeStruct((B,), jnp.float32),
    scratch_types=[
        pltpu.VMEM((IDXW,), jnp.int32),
        pltpu.VMEM((IDXW,), jnp.float32),
        pltpu.VMEM((RPW,), jnp.float32),
        pltpu.SemaphoreType.DMA,
    ],
)
def _sc_gather_sum(w_hbm, idx_hbm, out_hbm, idx_v, vals_v, acc_v, sem):
    wid = lax.axis_index("s") * NC + lax.axis_index("c")
    base = wid * IDXW
    pltpu.sync_copy(idx_hbm.at[pl.ds(base, IDXW)], idx_v)
    pltpu.async_copy(w_hbm.at[idx_v], vals_v, sem).wait()
    for c in range(RPW // LANES):
        col = c * LANES
        a = vals_v[pl.ds(col, LANES)]
        for j in range(1, F):
            a = a + vals_v[pl.ds(j * RPW + col, LANES)]
        acc_v[pl.ds(col, LANES)] = a
    pltpu.sync_copy(acc_v, out_hbm.at[pl.ds(wid * RPW, RPW)])


def _copy_body(i_ref, o_ref):
    o_ref[...] = i_ref[...]


_WROWS = 7936
_bar_copy = pl.pallas_call(
    _copy_body,
    grid=(16,),
    in_specs=[pl.BlockSpec((_WROWS // 16, 128), lambda i: (i, 0))],
    out_specs=pl.BlockSpec((_WROWS // 16, 128), lambda i: (i, 0)),
    out_shape=jax.ShapeDtypeStruct((_WROWS, 128), jnp.float32),
)


def _tc_body(b_ref, xw_ref, y_ref, yprob_ref, loss_ref):
    logits = xw_ref[...] + b_ref[0, 0]
    yprob_ref[...] = 1.0 / (1.0 + jnp.exp(-logits))
    ce = (
        jnp.maximum(logits, 0.0)
        - logits * y_ref[...]
        + jnp.log(1.0 + jnp.exp(-jnp.abs(logits)))
    )
    loss_ref[0, 0] = jnp.sum(ce) * (1.0 / B)


_tc_tail = pl.pallas_call(
    _tc_body,
    out_shape=(
        jax.ShapeDtypeStruct((B // 128, 128), jnp.float32),
        jax.ShapeDtypeStruct((1, 1), jnp.float32),
    ),
    in_specs=[
        pl.BlockSpec(memory_space=pltpu.SMEM),
        pl.BlockSpec(),
        pl.BlockSpec(),
    ],
    out_specs=(
        pl.BlockSpec(),
        pl.BlockSpec(memory_space=pltpu.SMEM),
    ),
)


def kernel(indices, values, y, w, b):
    idx_perm = indices.reshape(NW, RPW, F).transpose(0, 2, 1).reshape(-1)
    wp = jnp.pad(w, ((0, _WROWS * 128 - 1000000), (0, 0))).reshape(_WROWS, 128)
    xw = _sc_gather_sum(_bar_copy(wp).reshape(-1), idx_perm)
    yprob, loss = _tc_tail(
        b.reshape(1, 1), xw.reshape(B // 128, 128), y.reshape(B // 128, 128)
    )
    return yprob.reshape(-1), loss[0, 0]
